# baseline (speedup 1.0000x reference)
import dataclasses
import functools
import jax
import jax.numpy as jnp
from jax import lax
from jax.experimental import pallas as pl
from jax.experimental.pallas import tpu as pltpu
from jax.experimental.pallas import tpu_sc as plsc

D_FF = 11008
D_MODEL = 4096
NC, NS, L = 2, 16, 16
NW = NC * NS
NB = 8
UNROLL = 4

SPLIT = 6400
BK = 640
REST = D_FF - SPLIT
CAP = REST // NW
IDXCAP = CAP + 2 * NB

_mesh = plsc.VectorSubcoreMesh(core_axis_name="c", subcore_axis_name="s",
                               num_cores=NC, num_subcores=NS)

_sc_params = pltpu.CompilerParams()
if "needs_layout_passes" in pltpu.CompilerParams.__dataclass_fields__:
    _sc_params = dataclasses.replace(_sc_params, needs_layout_passes=False)


@functools.partial(
    pl.kernel,
    out_type=jax.ShapeDtypeStruct((NW, D_MODEL), jnp.float32),
    mesh=_mesh,
    scratch_types=[
        pltpu.VMEM((CAP,), jnp.float32),
        pltpu.VMEM((CAP,), jnp.float32),
        pltpu.VMEM((L,), jnp.float32),
        pltpu.VMEM((IDXCAP,), jnp.int32),
        pltpu.VMEM((IDXCAP,), jnp.float32),
        pltpu.VMEM((D_MODEL,), jnp.float32),
        pltpu.VMEM((NB, D_MODEL), jnp.float32),
        pltpu.VMEM((NB, D_MODEL), jnp.float32),
        pltpu.SemaphoreType.DMA,
        pltpu.SemaphoreType.DMA,
    ],
    compiler_params=_sc_params,
)
def _sc_masked_gemv(x_hbm, wn_hbm, t_hbm, w_hbm, part_hbm,
                    xv, wnv, tvv, idxb, xcv, acc, rows0, rows1, sem0, sem1):
    cid = lax.axis_index("c")
    sid = lax.axis_index("s")
    wid = cid * NS + sid
    base = SPLIT + wid * CAP

    pltpu.sync_copy(x_hbm.at[pl.ds(base, CAP)], xv)
    pltpu.sync_copy(wn_hbm.at[pl.ds(base, CAP)], wnv)
    pltpu.sync_copy(t_hbm, tvv)
    tv = tvv[...]
    lanes = lax.iota(jnp.int32, L)

    def comp_body(c, pos):
        xc = xv[pl.ds(c * L, L)]
        wc = wnv[pl.ds(c * L, L)]
        gidx = base + c * L + lanes
        m = jnp.abs(xc) * wc >= tv
        mi = m.astype(jnp.int32)
        offs = plsc.cumsum(mi) - 1 + pos
        plsc.store_scatter(idxb, [offs], gidx, mask=m)
        plsc.store_scatter(xcv, [offs], xc, mask=m)
        return pos + jnp.sum(mi)

    pos = lax.fori_loop(0, CAP // L, comp_body, jnp.int32(0))

    padpos = pos + lanes
    mpad = padpos < IDXCAP
    plsc.store_scatter(idxb, [padpos],
                       jnp.full((L,), SPLIT, jnp.int32), mask=mpad)
    plsc.store_scatter(xcv, [padpos],
                       jnp.zeros((L,), jnp.float32), mask=mpad)
    npair = (jnp.maximum(pos, 1) + (2 * NB - 1)) // (2 * NB)

    zero = jnp.zeros((L,), jnp.float32)

    @pl.loop(0, D_MODEL // L)
    def _(c):
        acc[pl.ds(c * L, L)] = zero

    def start(b, rows, sem):
        pltpu.async_copy(w_hbm.at[idxb.at[pl.ds(b * NB, NB)]], rows, sem)

    def wait(rows, sem):
        pltpu.make_async_copy(w_hbm.at[idxb.at[pl.ds(0, NB)]],
                              rows, sem).wait()

    def accum(b, rows):
        xv16 = xcv[pl.ds(b * NB, L)]
        wjs = [jnp.sum(jnp.where(lanes == j, xv16, jnp.float32(0.0)))
               for j in range(NB)]

        @pl.loop(0, D_MODEL // L, step=UNROLL)
        def _(c0):
            for u in range(UNROLL):
                sl = pl.ds((c0 + u) * L, L)
                terms = [wjs[j] * rows[j, sl] for j in range(NB)]
                while len(terms) > 1:
                    nxt = [terms[k] + terms[k + 1]
                           for k in range(0, len(terms) - 1, 2)]
                    if len(terms) % 2:
                        nxt.append(terms[-1])
                    terms = nxt
                acc[sl] += terms[0]

    start(0, rows0, sem0)

    def pair_body(i, carry):
        b0 = 2 * i
        start(b0 + 1, rows1, sem1)
        wait(rows0, sem0)
        accum(b0, rows0)

        @pl.when(i + 1 < npair)
        def _():
            start(b0 + 2, rows0, sem0)

        wait(rows1, sem1)
        accum(b0 + 1, rows1)
        return carry

    lax.fori_loop(0, npair, pair_body, jnp.int32(0))

    pltpu.sync_copy(acc, part_hbm.at[wid])


def _tc_body(t_ref, x_ref, wn_ref, o_ref, w_ref, y_ref):
    i = pl.program_id(0)
    xb = x_ref[...]
    m = jnp.abs(xb) * wn_ref[...] >= t_ref[0]
    xm = jnp.where(m, xb, jnp.float32(0.0))
    part = jnp.sum(w_ref[...] * xm, axis=0, keepdims=True)

    @pl.when(i == 0)
    def _():
        y_ref[...] = o_ref[...] + part

    @pl.when(i > 0)
    def _():
        y_ref[...] += part


def _combine_body(t_ref, p_ref, y_ref):
    y_ref[...] = t_ref[...] + jnp.sum(p_ref[...], axis=0, keepdims=True)


def kernel(x, W_t, W_norm, thresh, out):
    xf = x.reshape(-1)
    t16 = jnp.full((L,), thresh, jnp.float32)
    partials = _sc_masked_gemv(xf, W_norm, t16, W_t)

    t1 = jnp.reshape(thresh, (1,))
    tc_y = pl.pallas_call(
        _tc_body,
        grid=(SPLIT // BK,),
        in_specs=[
            pl.BlockSpec(memory_space=pltpu.SMEM),
            pl.BlockSpec((BK, 1), lambda i: (i, 0)),
            pl.BlockSpec((BK, 1), lambda i: (i, 0)),
            pl.BlockSpec((1, D_MODEL), lambda i: (0, 0)),
            pl.BlockSpec((BK, D_MODEL), lambda i: (i, 0)),
        ],
        out_specs=pl.BlockSpec((1, D_MODEL), lambda i: (0, 0)),
        out_shape=jax.ShapeDtypeStruct((1, D_MODEL), jnp.float32),
        compiler_params=pltpu.CompilerParams(
            dimension_semantics=("arbitrary",),
        ),
    )(t1, xf.reshape(D_FF, 1), W_norm.reshape(D_FF, 1),
      out.reshape(1, D_MODEL), W_t)

    y = pl.pallas_call(
        _combine_body,
        out_shape=jax.ShapeDtypeStruct((1, D_MODEL), jnp.float32),
    )(tc_y, partials)
    return y.reshape(D_MODEL)


# speedup vs baseline: 1.0514x; 1.0514x over previous
import dataclasses
import functools
import jax
import jax.numpy as jnp
from jax import lax
from jax.experimental import pallas as pl
from jax.experimental.pallas import tpu as pltpu
from jax.experimental.pallas import tpu_sc as plsc

D_FF = 11008
D_MODEL = 4096
NC, NS, L = 2, 16, 16
NW = NC * NS
NB = 8
UNROLL = 4

SPLIT = 6400
BK = 640
REST = D_FF - SPLIT
CAP = REST // NW
IDXCAP = CAP + 2 * NB

_mesh = plsc.VectorSubcoreMesh(core_axis_name="c", subcore_axis_name="s",
                               num_cores=NC, num_subcores=NS)

_sc_params = pltpu.CompilerParams()
if "needs_layout_passes" in pltpu.CompilerParams.__dataclass_fields__:
    _sc_params = dataclasses.replace(_sc_params, needs_layout_passes=False)


@functools.partial(
    pl.kernel,
    out_type=jax.ShapeDtypeStruct((NW, D_MODEL), jnp.float32),
    mesh=_mesh,
    scratch_types=[
        pltpu.VMEM((CAP,), jnp.float32),
        pltpu.VMEM((CAP,), jnp.float32),
        pltpu.VMEM((L,), jnp.float32),
        pltpu.VMEM((IDXCAP,), jnp.int32),
        pltpu.VMEM((IDXCAP,), jnp.float32),
        pltpu.VMEM((D_MODEL,), jnp.float32),
        pltpu.VMEM((NB, D_MODEL), jnp.float32),
        pltpu.VMEM((NB, D_MODEL), jnp.float32),
        pltpu.SemaphoreType.DMA,
        pltpu.SemaphoreType.DMA,
    ],
    compiler_params=_sc_params,
)
def _sc_masked_gemv(x_hbm, wn_hbm, t_hbm, w_hbm, part_hbm,
                    xv, wnv, tvv, idxb, xcv, acc, rows0, rows1, sem0, sem1):
    cid = lax.axis_index("c")
    sid = lax.axis_index("s")
    wid = cid * NS + sid
    base = SPLIT + wid * CAP

    pltpu.sync_copy(x_hbm.at[pl.ds(base, CAP)], xv)
    pltpu.sync_copy(wn_hbm.at[pl.ds(base, CAP)], wnv)
    pltpu.sync_copy(t_hbm, tvv)
    tv = tvv[...]
    lanes = lax.iota(jnp.int32, L)

    def comp_body(c, pos):
        xc = xv[pl.ds(c * L, L)]
        wc = wnv[pl.ds(c * L, L)]
        gidx = base + c * L + lanes
        m = jnp.abs(xc) * wc >= tv
        mi = m.astype(jnp.int32)
        offs = plsc.cumsum(mi) - 1 + pos
        plsc.store_scatter(idxb, [offs], gidx, mask=m)
        plsc.store_scatter(xcv, [offs], xc, mask=m)
        return pos + jnp.sum(mi)

    pos = lax.fori_loop(0, CAP // L, comp_body, jnp.int32(0))

    padpos = pos + lanes
    mpad = padpos < IDXCAP
    plsc.store_scatter(idxb, [padpos],
                       jnp.full((L,), SPLIT, jnp.int32), mask=mpad)
    plsc.store_scatter(xcv, [padpos],
                       jnp.zeros((L,), jnp.float32), mask=mpad)
    npair = (jnp.maximum(pos, 1) + (2 * NB - 1)) // (2 * NB)

    zero = jnp.zeros((L,), jnp.float32)

    @pl.loop(0, D_MODEL // L)
    def _(c):
        acc[pl.ds(c * L, L)] = zero

    def start(b, rows, sem):
        pltpu.async_copy(w_hbm.at[idxb.at[pl.ds(b * NB, NB)]], rows, sem)

    def wait(rows, sem):
        pltpu.make_async_copy(w_hbm.at[idxb.at[pl.ds(0, NB)]],
                              rows, sem).wait()

    def accum(b, rows):
        xv16 = xcv[pl.ds(b * NB, L)]
        wjs = [jnp.sum(jnp.where(lanes == j, xv16, jnp.float32(0.0)))
               for j in range(NB)]

        @pl.loop(0, D_MODEL // L, step=UNROLL)
        def _(c0):
            for u in range(UNROLL):
                sl = pl.ds((c0 + u) * L, L)
                terms = [wjs[j] * rows[j, sl] for j in range(NB)]
                while len(terms) > 1:
                    nxt = [terms[k] + terms[k + 1]
                           for k in range(0, len(terms) - 1, 2)]
                    if len(terms) % 2:
                        nxt.append(terms[-1])
                    terms = nxt
                acc[sl] += terms[0]

    start(0, rows0, sem0)

    def pair_body(i, carry):
        b0 = 2 * i
        start(b0 + 1, rows1, sem1)
        wait(rows0, sem0)
        accum(b0, rows0)

        @pl.when(i + 1 < npair)
        def _():
            start(b0 + 2, rows0, sem0)

        wait(rows1, sem1)
        accum(b0 + 1, rows1)
        return carry

    lax.fori_loop(0, npair, pair_body, jnp.int32(0))

    pltpu.sync_copy(acc, part_hbm.at[wid])


def _tc_body(t_ref, x_ref, wn_ref, o_ref, w_ref, y_ref):
    i = pl.program_id(0)
    xb = x_ref[...]
    m = jnp.abs(xb) * wn_ref[...] >= t_ref[0]
    xm = jnp.where(m, xb, jnp.float32(0.0))
    xm_col = jnp.reshape(xm, (BK, 1))
    part = jnp.sum(w_ref[...] * xm_col, axis=0, keepdims=True)

    @pl.when(i == 0)
    def _():
        y_ref[...] = o_ref[...] + part

    @pl.when(i > 0)
    def _():
        y_ref[...] += part


def _combine_body(t_ref, p_ref, y_ref):
    y_ref[...] = t_ref[...] + jnp.sum(p_ref[...], axis=0, keepdims=True)


def kernel(x, W_t, W_norm, thresh, out):
    xf = x.reshape(-1)
    t16 = jnp.full((L,), thresh, jnp.float32)
    partials = _sc_masked_gemv(xf, W_norm, t16, W_t)

    t1 = jnp.reshape(thresh, (1,))
    tc_y = pl.pallas_call(
        _tc_body,
        grid=(SPLIT // BK,),
        in_specs=[
            pl.BlockSpec(memory_space=pltpu.SMEM),
            pl.BlockSpec((1, BK), lambda i: (0, i)),
            pl.BlockSpec((1, BK), lambda i: (0, i)),
            pl.BlockSpec((1, D_MODEL), lambda i: (0, 0)),
            pl.BlockSpec((BK, D_MODEL), lambda i: (i, 0)),
        ],
        out_specs=pl.BlockSpec((1, D_MODEL), lambda i: (0, 0)),
        out_shape=jax.ShapeDtypeStruct((1, D_MODEL), jnp.float32),
        compiler_params=pltpu.CompilerParams(
            dimension_semantics=("arbitrary",),
        ),
    )(t1, xf.reshape(1, D_FF), W_norm.reshape(1, D_FF),
      out.reshape(1, D_MODEL), W_t)

    y = pl.pallas_call(
        _combine_body,
        out_shape=jax.ShapeDtypeStruct((1, D_MODEL), jnp.float32),
    )(tc_y, partials)
    return y.reshape(D_MODEL)


# speedup vs baseline: 1.0856x; 1.0325x over previous
import dataclasses
import functools
import jax
import jax.numpy as jnp
from jax import lax
from jax.experimental import pallas as pl
from jax.experimental.pallas import tpu as pltpu
from jax.experimental.pallas import tpu_sc as plsc

D_FF = 11008
D_MODEL = 4096
NC, NS, L = 2, 16, 16
NW = NC * NS
NB = 8
UNROLL = 4

SPLIT = 6912
BK = 768
REST = D_FF - SPLIT
CAP = REST // NW
IDXCAP = CAP + 2 * NB

_mesh = plsc.VectorSubcoreMesh(core_axis_name="c", subcore_axis_name="s",
                               num_cores=NC, num_subcores=NS)

_sc_params = pltpu.CompilerParams()
if "needs_layout_passes" in pltpu.CompilerParams.__dataclass_fields__:
    _sc_params = dataclasses.replace(_sc_params, needs_layout_passes=False)


@functools.partial(
    pl.kernel,
    out_type=jax.ShapeDtypeStruct((NW, D_MODEL), jnp.float32),
    mesh=_mesh,
    scratch_types=[
        pltpu.VMEM((CAP,), jnp.float32),
        pltpu.VMEM((CAP,), jnp.float32),
        pltpu.VMEM((L,), jnp.float32),
        pltpu.VMEM((IDXCAP,), jnp.int32),
        pltpu.VMEM((IDXCAP,), jnp.float32),
        pltpu.VMEM((D_MODEL,), jnp.float32),
        pltpu.VMEM((NB, D_MODEL), jnp.float32),
        pltpu.VMEM((NB, D_MODEL), jnp.float32),
        pltpu.SemaphoreType.DMA,
        pltpu.SemaphoreType.DMA,
    ],
    compiler_params=_sc_params,
)
def _sc_masked_gemv(x_hbm, wn_hbm, t_hbm, w_hbm, part_hbm,
                    xv, wnv, tvv, idxb, xcv, acc, rows0, rows1, sem0, sem1):
    cid = lax.axis_index("c")
    sid = lax.axis_index("s")
    wid = cid * NS + sid
    base = SPLIT + wid * CAP

    pltpu.sync_copy(x_hbm.at[pl.ds(base, CAP)], xv)
    pltpu.sync_copy(wn_hbm.at[pl.ds(base, CAP)], wnv)
    pltpu.sync_copy(t_hbm, tvv)
    tv = tvv[...]
    lanes = lax.iota(jnp.int32, L)

    def comp_body(c, pos):
        xc = xv[pl.ds(c * L, L)]
        wc = wnv[pl.ds(c * L, L)]
        gidx = base + c * L + lanes
        m = jnp.abs(xc) * wc >= tv
        mi = m.astype(jnp.int32)
        offs = plsc.cumsum(mi) - 1 + pos
        plsc.store_scatter(idxb, [offs], gidx, mask=m)
        plsc.store_scatter(xcv, [offs], xc, mask=m)
        return pos + jnp.sum(mi)

    pos = lax.fori_loop(0, CAP // L, comp_body, jnp.int32(0))

    padpos = pos + lanes
    mpad = padpos < IDXCAP
    plsc.store_scatter(idxb, [padpos],
                       jnp.full((L,), SPLIT, jnp.int32), mask=mpad)
    plsc.store_scatter(xcv, [padpos],
                       jnp.zeros((L,), jnp.float32), mask=mpad)
    npair = (jnp.maximum(pos, 1) + (2 * NB - 1)) // (2 * NB)

    zero = jnp.zeros((L,), jnp.float32)

    @pl.loop(0, D_MODEL // L)
    def _(c):
        acc[pl.ds(c * L, L)] = zero

    def start(b, rows, sem):
        pltpu.async_copy(w_hbm.at[idxb.at[pl.ds(b * NB, NB)]], rows, sem)

    def wait(rows, sem):
        pltpu.make_async_copy(w_hbm.at[idxb.at[pl.ds(0, NB)]],
                              rows, sem).wait()

    def accum(b, rows):
        xv16 = xcv[pl.ds(b * NB, L)]
        wjs = [jnp.sum(jnp.where(lanes == j, xv16, jnp.float32(0.0)))
               for j in range(NB)]

        @pl.loop(0, D_MODEL // L, step=UNROLL)
        def _(c0):
            for u in range(UNROLL):
                sl = pl.ds((c0 + u) * L, L)
                terms = [wjs[j] * rows[j, sl] for j in range(NB)]
                while len(terms) > 1:
                    nxt = [terms[k] + terms[k + 1]
                           for k in range(0, len(terms) - 1, 2)]
                    if len(terms) % 2:
                        nxt.append(terms[-1])
                    terms = nxt
                acc[sl] += terms[0]

    start(0, rows0, sem0)

    def pair_body(i, carry):
        b0 = 2 * i
        start(b0 + 1, rows1, sem1)
        wait(rows0, sem0)
        accum(b0, rows0)

        @pl.when(i + 1 < npair)
        def _():
            start(b0 + 2, rows0, sem0)

        wait(rows1, sem1)
        accum(b0 + 1, rows1)
        return carry

    lax.fori_loop(0, npair, pair_body, jnp.int32(0))

    pltpu.sync_copy(acc, part_hbm.at[wid])


def _tc_body(t_ref, x_ref, wn_ref, o_ref, w_ref, y_ref):
    i = pl.program_id(0)
    xb = x_ref[...]
    m = jnp.abs(xb) * wn_ref[...] >= t_ref[0]
    xm = jnp.where(m, xb, jnp.float32(0.0))
    xm_col = jnp.reshape(xm, (BK, 1))
    part = jnp.sum(w_ref[...] * xm_col, axis=0, keepdims=True)

    @pl.when(i == 0)
    def _():
        y_ref[...] = o_ref[...] + part

    @pl.when(i > 0)
    def _():
        y_ref[...] += part


def _combine_body(t_ref, p_ref, y_ref):
    y_ref[...] = t_ref[...] + jnp.sum(p_ref[...], axis=0, keepdims=True)


def kernel(x, W_t, W_norm, thresh, out):
    xf = x.reshape(-1)
    t16 = jnp.full((L,), thresh, jnp.float32)
    partials = _sc_masked_gemv(xf, W_norm, t16, W_t)

    t1 = jnp.reshape(thresh, (1,))
    tc_y = pl.pallas_call(
        _tc_body,
        grid=(SPLIT // BK,),
        in_specs=[
            pl.BlockSpec(memory_space=pltpu.SMEM),
            pl.BlockSpec((1, BK), lambda i: (0, i)),
            pl.BlockSpec((1, BK), lambda i: (0, i)),
            pl.BlockSpec((1, D_MODEL), lambda i: (0, 0)),
            pl.BlockSpec((BK, D_MODEL), lambda i: (i, 0)),
        ],
        out_specs=pl.BlockSpec((1, D_MODEL), lambda i: (0, 0)),
        out_shape=jax.ShapeDtypeStruct((1, D_MODEL), jnp.float32),
        compiler_params=pltpu.CompilerParams(
            dimension_semantics=("arbitrary",),
        ),
    )(t1, xf.reshape(1, D_FF), W_norm.reshape(1, D_FF),
      out.reshape(1, D_MODEL), W_t)

    y = pl.pallas_call(
        _combine_body,
        out_shape=jax.ShapeDtypeStruct((1, D_MODEL), jnp.float32),
    )(tc_y, partials)
    return y.reshape(D_MODEL)


# speedup vs baseline: 1.1050x; 1.0179x over previous
import dataclasses
import functools
import jax
import jax.numpy as jnp
from jax import lax
from jax.experimental import pallas as pl
from jax.experimental.pallas import tpu as pltpu
from jax.experimental.pallas import tpu_sc as plsc

D_FF = 11008
D_MODEL = 4096
NC, NS, L = 2, 16, 16
NW = NC * NS
NB = 8
UNROLL = 4

SPLIT = 6912
BK = 768
REST = D_FF - SPLIT
CAP = REST // NW
IDXCAP = CAP + 2 * NB

_mesh = plsc.VectorSubcoreMesh(core_axis_name="c", subcore_axis_name="s",
                               num_cores=NC, num_subcores=NS)

_sc_params = pltpu.CompilerParams()
if "needs_layout_passes" in pltpu.CompilerParams.__dataclass_fields__:
    _sc_params = dataclasses.replace(_sc_params, needs_layout_passes=False)


@functools.partial(
    pl.kernel,
    out_type=jax.ShapeDtypeStruct((NW, D_MODEL), jnp.float32),
    mesh=_mesh,
    scratch_types=[
        pltpu.VMEM((CAP,), jnp.float32),
        pltpu.VMEM((CAP,), jnp.float32),
        pltpu.VMEM((L,), jnp.float32),
        pltpu.VMEM((IDXCAP,), jnp.int32),
        pltpu.VMEM((IDXCAP,), jnp.float32),
        pltpu.VMEM((D_MODEL,), jnp.float32),
        pltpu.VMEM((NB, D_MODEL), jnp.float32),
        pltpu.VMEM((NB, D_MODEL), jnp.float32),
        pltpu.SemaphoreType.DMA,
        pltpu.SemaphoreType.DMA,
    ],
    compiler_params=_sc_params,
)
def _sc_masked_gemv(x_hbm, wn_hbm, t_hbm, w_hbm, part_hbm,
                    xv, wnv, tvv, idxb, xcv, acc, rows0, rows1, sem0, sem1):
    cid = lax.axis_index("c")
    sid = lax.axis_index("s")
    wid = cid * NS + sid
    base = SPLIT + wid * CAP

    pltpu.sync_copy(x_hbm.at[pl.ds(base, CAP)], xv)
    pltpu.sync_copy(wn_hbm.at[pl.ds(base, CAP)], wnv)
    pltpu.sync_copy(t_hbm, tvv)
    tv = tvv[...]
    lanes = lax.iota(jnp.int32, L)

    def comp_body(c, pos):
        xc = xv[pl.ds(c * L, L)]
        wc = wnv[pl.ds(c * L, L)]
        gidx = base + c * L + lanes
        m = jnp.abs(xc) * wc >= tv
        mi = m.astype(jnp.int32)
        offs = plsc.cumsum(mi) - 1 + pos
        plsc.store_scatter(idxb, [offs], gidx, mask=m)
        plsc.store_scatter(xcv, [offs], xc, mask=m)
        return pos + jnp.sum(mi)

    pos = lax.fori_loop(0, CAP // L, comp_body, jnp.int32(0))

    padpos = pos + lanes
    mpad = padpos < IDXCAP
    plsc.store_scatter(idxb, [padpos],
                       jnp.full((L,), SPLIT, jnp.int32), mask=mpad)
    plsc.store_scatter(xcv, [padpos],
                       jnp.zeros((L,), jnp.float32), mask=mpad)
    npair = (jnp.maximum(pos, 1) + (2 * NB - 1)) // (2 * NB)

    zero = jnp.zeros((L,), jnp.float32)

    @pl.loop(0, D_MODEL // L)
    def _(c):
        acc[pl.ds(c * L, L)] = zero

    def start(b, rows, sem):
        pltpu.async_copy(w_hbm.at[idxb.at[pl.ds(b * NB, NB)]], rows, sem)

    def wait(rows, sem):
        pltpu.make_async_copy(w_hbm.at[idxb.at[pl.ds(0, NB)]],
                              rows, sem).wait()

    def accum(b, rows):
        xv16 = xcv[pl.ds(b * NB, L)]
        wjs = [jnp.sum(jnp.where(lanes == j, xv16, jnp.float32(0.0)))
               for j in range(NB)]

        @pl.loop(0, D_MODEL // L, step=UNROLL)
        def _(c0):
            for u in range(UNROLL):
                sl = pl.ds((c0 + u) * L, L)
                terms = [wjs[j] * rows[j, sl] for j in range(NB)]
                while len(terms) > 1:
                    nxt = [terms[k] + terms[k + 1]
                           for k in range(0, len(terms) - 1, 2)]
                    if len(terms) % 2:
                        nxt.append(terms[-1])
                    terms = nxt
                acc[sl] += terms[0]

    start(0, rows0, sem0)

    def pair_body(i, carry):
        b0 = 2 * i
        start(b0 + 1, rows1, sem1)
        wait(rows0, sem0)
        accum(b0, rows0)

        @pl.when(i + 1 < npair)
        def _():
            start(b0 + 2, rows0, sem0)

        wait(rows1, sem1)
        accum(b0 + 1, rows1)
        return carry

    lax.fori_loop(0, npair, pair_body, jnp.int32(0))

    pltpu.sync_copy(acc, part_hbm.at[wid])


def _tc_body(t_ref, x_ref, wn_ref, o_ref, w_ref, y_ref):
    i = pl.program_id(0)
    xb = x_ref[...]
    m = jnp.abs(xb) * wn_ref[...] >= t_ref[0]
    xm = jnp.where(m, xb, jnp.float32(0.0))
    xm_col = jnp.reshape(xm, (BK, 1))
    part = jnp.sum(w_ref[...] * xm_col, axis=0, keepdims=True)

    @pl.when(i == 0)
    def _():
        y_ref[...] = o_ref[...] + part

    @pl.when(i > 0)
    def _():
        y_ref[...] += part


def _combine_body(t_ref, p_ref, y_ref):
    y_ref[...] = t_ref[...] + jnp.sum(p_ref[...], axis=0, keepdims=True)


def kernel(x, W_t, W_norm, thresh, out):
    xf = x.reshape(-1)
    t1 = jnp.reshape(thresh, (1,))
    tc_y = pl.pallas_call(
        _tc_body,
        grid=(SPLIT // BK,),
        in_specs=[
            pl.BlockSpec(memory_space=pltpu.SMEM),
            pl.BlockSpec((1, BK), lambda i: (0, i)),
            pl.BlockSpec((1, BK), lambda i: (0, i)),
            pl.BlockSpec((1, D_MODEL), lambda i: (0, 0)),
            pl.BlockSpec((BK, D_MODEL), lambda i: (i, 0)),
        ],
        out_specs=pl.BlockSpec((1, D_MODEL), lambda i: (0, 0)),
        out_shape=jax.ShapeDtypeStruct((1, D_MODEL), jnp.float32),
        compiler_params=pltpu.CompilerParams(
            dimension_semantics=("arbitrary",),
        ),
    )(t1, xf.reshape(1, D_FF), W_norm.reshape(1, D_FF),
      out.reshape(1, D_MODEL), W_t)

    t16 = jnp.full((L,), thresh, jnp.float32)
    partials = _sc_masked_gemv(xf, W_norm, t16, W_t)

    y = pl.pallas_call(
        _combine_body,
        out_shape=jax.ShapeDtypeStruct((1, D_MODEL), jnp.float32),
    )(tc_y, partials)
    return y.reshape(D_MODEL)
